# baseline (speedup 1.0000x reference)
import functools

import jax
import jax.numpy as jnp
from jax import lax
from jax.experimental import pallas as pl
from jax.experimental.pallas import tpu as pltpu
from jax.experimental.pallas import tpu_sc as plsc

N_REGIONS = 6
BOUND = 2.5

_B, _C, _H, _W = 8, 96, 224, 224
_N = _B * _C * _H * _W
_ROW = _H * _W
_NW = 32
_PER_W = _N // _NW
_CHUNK = _ROW // 2
_CHUNKS_PER_W = _PER_W // _CHUNK
_ROWS_PER_W = _PER_W // _ROW
_VECS = _CHUNK // 16
_TAB = _B * _C * N_REGIONS


def _sc_body(x_hbm, a_hbm, b_hbm, out_hbm, in_v, out_v, a_tab, b_tab):
    wid = lax.axis_index("s") * 2 + lax.axis_index("c")
    pltpu.sync_copy(a_hbm, a_tab)
    pltpu.sync_copy(b_hbm, b_tab)

    def chunk_body(k, _):
        off = wid * _PER_W + k * _CHUNK
        base = (wid * _ROWS_PER_W + k // 2) * N_REGIONS
        pltpu.sync_copy(x_hbm.at[pl.ds(off, _CHUNK)], in_v)

        def vec_body(i, _):
            v = in_v[pl.ds(i * 16, 16)]
            s = v * (0.5 * N_REGIONS / BOUND) + (0.5 * N_REGIONS)
            sc = jnp.minimum(jnp.maximum(s, 0.0), float(N_REGIONS) * 0.999)
            idx = sc.astype(jnp.int32) + base
            a = plsc.load_gather(a_tab, [idx])
            b = plsc.load_gather(b_tab, [idx])
            out_v[pl.ds(i * 16, 16)] = a + b * s
            return 0

        lax.fori_loop(0, _VECS, vec_body, 0)
        pltpu.sync_copy(out_v, out_hbm.at[pl.ds(off, _CHUNK)])
        return 0

    lax.fori_loop(0, _CHUNKS_PER_W, chunk_body, 0)


def kernel(x, points):
    B, C, H, W = x.shape

    slopes = points[:, 1:] - points[:, :-1]
    intercepts = points[:, :-1] - slopes * jnp.arange(
        N_REGIONS, dtype=points.dtype
    )[None, :]
    a_flat = jnp.tile(intercepts, (B, 1)).reshape(-1)
    b_flat = jnp.tile(slopes, (B, 1)).reshape(-1)

    xf = x.reshape(-1)
    sc_kernel = functools.partial(
        pl.kernel,
        out_type=jax.ShapeDtypeStruct((_N,), jnp.float32),
        mesh=plsc.VectorSubcoreMesh(core_axis_name="c", subcore_axis_name="s"),
        compiler_params=pltpu.CompilerParams(needs_layout_passes=False),
        scratch_types=[
            pltpu.VMEM((_CHUNK,), jnp.float32),
            pltpu.VMEM((_CHUNK,), jnp.float32),
            pltpu.VMEM((_TAB,), jnp.float32),
            pltpu.VMEM((_TAB,), jnp.float32),
        ],
    )(_sc_body)
    out = sc_kernel(xf, a_flat, b_flat)
    return out.reshape(B, C, H, W)


# speedup vs baseline: 1.6251x; 1.6251x over previous
import functools

import jax
import jax.numpy as jnp
from jax import lax
from jax.experimental import pallas as pl
from jax.experimental.pallas import tpu as pltpu
from jax.experimental.pallas import tpu_sc as plsc

N_REGIONS = 6
BOUND = 2.5

_B, _C, _H, _W = 8, 96, 224, 224
_N = _B * _C * _H * _W
_ROW = _H * _W
_NW = 32
_PER_W = _N // _NW
_CHUNK = _ROW // 2
_CHUNKS_PER_W = _PER_W // _CHUNK
_ROWS_PER_W = _PER_W // _ROW
_VECS = _CHUNK // 16
_TAB = _B * _C * N_REGIONS


def _sc_body(x_hbm, a_hbm, b_hbm, out_hbm, in_v, out_v, a_tab, b_tab):
    wid = lax.axis_index("s") * 2 + lax.axis_index("c")
    pltpu.sync_copy(a_hbm, a_tab)
    pltpu.sync_copy(b_hbm, b_tab)

    def chunk_body(k, _):
        off = wid * _PER_W + k * _CHUNK
        base = (wid * _ROWS_PER_W + k // 2) * N_REGIONS
        pltpu.sync_copy(x_hbm.at[pl.ds(off, _CHUNK)], in_v)

        @plsc.parallel_loop(0, _CHUNK, step=16, unroll=8)
        def vec_body(i):
            v = in_v[pl.ds(i, 16)]
            s = v * (0.5 * N_REGIONS / BOUND) + (0.5 * N_REGIONS)
            sc = jnp.minimum(jnp.maximum(s, 0.0), float(N_REGIONS) * 0.999)
            idx = sc.astype(jnp.int32) + base
            a = plsc.load_gather(a_tab, [idx])
            b = plsc.load_gather(b_tab, [idx])
            out_v[pl.ds(i, 16)] = a + b * s
        pltpu.sync_copy(out_v, out_hbm.at[pl.ds(off, _CHUNK)])
        return 0

    lax.fori_loop(0, _CHUNKS_PER_W, chunk_body, 0)


def kernel(x, points):
    B, C, H, W = x.shape

    slopes = points[:, 1:] - points[:, :-1]
    intercepts = points[:, :-1] - slopes * jnp.arange(
        N_REGIONS, dtype=points.dtype
    )[None, :]
    a_flat = jnp.tile(intercepts, (B, 1)).reshape(-1)
    b_flat = jnp.tile(slopes, (B, 1)).reshape(-1)

    xf = x.reshape(-1)
    sc_kernel = functools.partial(
        pl.kernel,
        out_type=jax.ShapeDtypeStruct((_N,), jnp.float32),
        mesh=plsc.VectorSubcoreMesh(core_axis_name="c", subcore_axis_name="s"),
        compiler_params=pltpu.CompilerParams(needs_layout_passes=False),
        scratch_types=[
            pltpu.VMEM((_CHUNK,), jnp.float32),
            pltpu.VMEM((_CHUNK,), jnp.float32),
            pltpu.VMEM((_TAB,), jnp.float32),
            pltpu.VMEM((_TAB,), jnp.float32),
        ],
    )(_sc_body)
    out = sc_kernel(xf, a_flat, b_flat)
    return out.reshape(B, C, H, W)
